# baseline (speedup 1.0000x reference)
import functools

import jax
import jax.numpy as jnp
from jax import lax
from jax.experimental import pallas as pl
from jax.experimental.pallas import tpu as pltpu
from jax.experimental.pallas import tpu_sc as plsc


def _sc_row_gather(table, idx, chunk):
    _, d = table.shape
    m = idx.shape[0]
    info = plsc.get_sparse_core_info()
    nw = info.num_cores * info.num_subcores
    rows_per_w = m // nw
    assert rows_per_w * nw == m
    n_chunks = rows_per_w // chunk
    assert n_chunks * chunk == rows_per_w
    assert chunk % 8 == 0 and chunk <= 128

    mesh = plsc.VectorSubcoreMesh(core_axis_name="c", subcore_axis_name="s")

    @functools.partial(
        pl.kernel,
        out_type=jax.ShapeDtypeStruct((m, d), table.dtype),
        mesh=mesh,
        scratch_types=[
            pltpu.VMEM((chunk,), jnp.int32),
            pltpu.VMEM((chunk, d), table.dtype),
            pltpu.SemaphoreType.DMA,
        ],
    )
    def gather_kernel(table_hbm, idx_hbm, out_hbm, idx_v, rows_v, sem):
        wid = lax.axis_index("s") * info.num_cores + lax.axis_index("c")
        base = wid * rows_per_w

        def body(c, carry):
            row0 = base + c * chunk
            pltpu.sync_copy(idx_hbm.at[pl.ds(row0, chunk)], idx_v)
            pltpu.async_copy(table_hbm.at[idx_v], rows_v, sem).wait()
            pltpu.sync_copy(rows_v, out_hbm.at[pl.ds(row0, chunk), :])
            return carry

        lax.fori_loop(0, n_chunks, body, 0)

    return gather_kernel(table, idx)


def _ffn_body(g_ref, x_ref, w1_ref, b1_ref, w2_ref, b2_ref, o_ref,
              *, kf_total):
    kf = pl.program_id(1)

    x = x_ref[...]
    h = jnp.dot(x, w1_ref[0], preferred_element_type=jnp.float32)
    h = jax.nn.gelu(h + b1_ref[0, 0][None, :])
    acc = jnp.dot(h, w2_ref[0], preferred_element_type=jnp.float32)
    last = (kf == kf_total - 1).astype(jnp.float32)
    acc = acc + last * b2_ref[0, 0][None, :]

    @pl.when(kf == 0)
    def _store():
        o_ref[...] = acc

    @pl.when(kf != 0)
    def _accum():
        o_ref[...] += acc


def _grouped_ffn(x_padded, w1, b1, w2, b2, g_ids, tm, tf, interpret=False):
    n_pad, d = x_padded.shape
    e, _, ff = w1.shape
    t_slots = g_ids.shape[0]
    kf_total = ff // tf
    assert t_slots * tm == n_pad

    grid_spec = pltpu.PrefetchScalarGridSpec(
        num_scalar_prefetch=1,
        grid=(t_slots, kf_total),
        in_specs=[
            pl.BlockSpec((tm, d), lambda t, kf, g: (t, 0)),
            pl.BlockSpec((1, d, tf), lambda t, kf, g: (g[t], 0, kf)),
            pl.BlockSpec((1, 1, tf), lambda t, kf, g: (g[t], 0, kf)),
            pl.BlockSpec((1, tf, d), lambda t, kf, g: (g[t], kf, 0)),
            pl.BlockSpec((1, 1, d), lambda t, kf, g: (g[t], 0, 0)),
        ],
        out_specs=pl.BlockSpec((tm, d), lambda t, kf, g: (t, 0)),
    )
    return pl.pallas_call(
        functools.partial(_ffn_body, kf_total=kf_total),
        grid_spec=grid_spec,
        out_shape=jax.ShapeDtypeStruct((n_pad, d), x_padded.dtype),
        compiler_params=pltpu.CompilerParams(
            dimension_semantics=("arbitrary", "arbitrary"),
        ),
        interpret=interpret,
    )(g_ids, x_padded, w1,
      b1.reshape(e, 1, ff), w2, b2.reshape(e, 1, d))


def _routing(tt, n, e, tm, t_slots):
    perm = jnp.argsort(tt).astype(jnp.int32)
    counts = jnp.sum(tt[None, :] == jnp.arange(e, dtype=jnp.int32)[:, None],
                     axis=1, dtype=jnp.int32)
    offsets = jnp.concatenate([jnp.zeros((1,), jnp.int32),
                               jnp.cumsum(counts).astype(jnp.int32)])
    ntiles = (counts + tm - 1) // tm
    base = jnp.concatenate([jnp.zeros((1,), jnp.int32),
                            jnp.cumsum(ntiles).astype(jnp.int32)])
    pad_off = base * tm

    ts = jnp.arange(t_slots, dtype=jnp.int32)
    g = jnp.searchsorted(base, ts, side="right").astype(jnp.int32) - 1
    g_ids = jnp.clip(g, 0, e - 1)

    i = jnp.arange(n, dtype=jnp.int32)
    e_i = jnp.searchsorted(offsets, i, side="right").astype(jnp.int32) - 1
    dest = pad_off[e_i] + (i - offsets[e_i])

    idx_pad = jnp.zeros((t_slots * tm,), jnp.int32).at[dest].set(perm)
    inv_idx = jnp.zeros((n,), jnp.int32).at[perm].set(dest)
    return g_ids, idx_pad, inv_idx


def kernel(hidden_states, token_type_ids, W1, b1, W2, b2):
    b, s, d = hidden_states.shape
    e, _, ff = W1.shape
    n = b * s
    tm, tf = 512, 1024
    t_slots = n // tm + e

    flat = hidden_states.reshape(n, d)
    tt = token_type_ids.reshape(n).astype(jnp.int32)

    g_ids, idx_pad, inv_idx = _routing(tt, n, e, tm, t_slots)

    x_padded = _sc_row_gather(flat, idx_pad, chunk=32)
    y_padded = _grouped_ffn(x_padded, W1, b1, W2, b2, g_ids, tm, tf)
    out = _sc_row_gather(y_padded, inv_idx, chunk=32)
    return out.reshape(b, s, d)


# speedup vs baseline: 1.1778x; 1.1778x over previous
import functools

import jax
import jax.numpy as jnp
from jax import lax
from jax.experimental import pallas as pl
from jax.experimental.pallas import tpu as pltpu
from jax.experimental.pallas import tpu_sc as plsc


def _sc_row_gather(table, idx, chunk):
    _, d = table.shape
    m = idx.shape[0]
    info = plsc.get_sparse_core_info()
    nw = info.num_cores * info.num_subcores
    rows_per_w = m // nw
    assert rows_per_w * nw == m
    n_chunks = rows_per_w // chunk
    assert n_chunks * chunk == rows_per_w
    assert chunk % 8 == 0 and chunk <= 128

    mesh = plsc.VectorSubcoreMesh(core_axis_name="c", subcore_axis_name="s")

    @functools.partial(
        pl.kernel,
        out_type=jax.ShapeDtypeStruct((m, d), table.dtype),
        mesh=mesh,
        scratch_types=[
            pltpu.VMEM((chunk,), jnp.int32),
            pltpu.VMEM((chunk, d), table.dtype),
            pltpu.SemaphoreType.DMA,
        ],
    )
    def gather_kernel(table_hbm, idx_hbm, out_hbm, idx_v, rows_v, sem):
        wid = lax.axis_index("s") * info.num_cores + lax.axis_index("c")
        base = wid * rows_per_w

        def body(c, carry):
            row0 = base + c * chunk
            pltpu.sync_copy(idx_hbm.at[pl.ds(row0, chunk)], idx_v)
            pltpu.async_copy(table_hbm.at[idx_v], rows_v, sem).wait()
            pltpu.sync_copy(rows_v, out_hbm.at[pl.ds(row0, chunk), :])
            return carry

        lax.fori_loop(0, n_chunks, body, 0)

    return gather_kernel(table, idx)


def _ffn_body(g_ref, x_ref, w1_ref, b1_ref, w2_ref, b2_ref, o_ref,
              *, kf_total):
    kf = pl.program_id(1)

    x = x_ref[...]
    h = jnp.dot(x, w1_ref[0], preferred_element_type=jnp.float32)
    h = jax.nn.gelu(h + b1_ref[0, 0][None, :])
    acc = jnp.dot(h, w2_ref[0], preferred_element_type=jnp.float32)
    last = (kf == kf_total - 1).astype(jnp.float32)
    acc = acc + last * b2_ref[0, 0][None, :]

    @pl.when(kf == 0)
    def _store():
        o_ref[...] = acc

    @pl.when(kf != 0)
    def _accum():
        o_ref[...] += acc


def _grouped_ffn(x_padded, w1, b1, w2, b2, g_ids, tm, tf, interpret=False):
    n_pad, d = x_padded.shape
    e, _, ff = w1.shape
    t_slots = g_ids.shape[0]
    kf_total = ff // tf
    assert t_slots * tm == n_pad

    grid_spec = pltpu.PrefetchScalarGridSpec(
        num_scalar_prefetch=1,
        grid=(t_slots, kf_total),
        in_specs=[
            pl.BlockSpec((tm, d), lambda t, kf, g: (t, 0)),
            pl.BlockSpec((1, d, tf), lambda t, kf, g: (g[t], 0, kf)),
            pl.BlockSpec((1, 1, tf), lambda t, kf, g: (g[t], 0, kf)),
            pl.BlockSpec((1, tf, d), lambda t, kf, g: (g[t], kf, 0)),
            pl.BlockSpec((1, 1, d), lambda t, kf, g: (g[t], 0, 0)),
        ],
        out_specs=pl.BlockSpec((tm, d), lambda t, kf, g: (t, 0)),
    )
    return pl.pallas_call(
        functools.partial(_ffn_body, kf_total=kf_total),
        grid_spec=grid_spec,
        out_shape=jax.ShapeDtypeStruct((n_pad, d), x_padded.dtype),
        compiler_params=pltpu.CompilerParams(
            dimension_semantics=("arbitrary", "arbitrary"),
        ),
        interpret=interpret,
    )(g_ids, x_padded, w1,
      b1.reshape(e, 1, ff), w2, b2.reshape(e, 1, d))


def _routing(tt, n, e, tm, t_slots):
    perm = jnp.argsort(tt).astype(jnp.int32)
    counts = jnp.sum(tt[None, :] == jnp.arange(e, dtype=jnp.int32)[:, None],
                     axis=1, dtype=jnp.int32)
    offsets = jnp.concatenate([jnp.zeros((1,), jnp.int32),
                               jnp.cumsum(counts).astype(jnp.int32)])
    ntiles = (counts + tm - 1) // tm
    base = jnp.concatenate([jnp.zeros((1,), jnp.int32),
                            jnp.cumsum(ntiles).astype(jnp.int32)])
    pad_off = base * tm

    ts = jnp.arange(t_slots, dtype=jnp.int32)
    g = jnp.searchsorted(base, ts, side="right").astype(jnp.int32) - 1
    g_ids = jnp.clip(g, 0, e - 1)

    i = jnp.arange(n, dtype=jnp.int32)
    e_i = jnp.searchsorted(offsets, i, side="right").astype(jnp.int32) - 1
    dest = pad_off[e_i] + (i - offsets[e_i])

    n_pad = t_slots * tm
    idx_pad = (jnp.arange(n_pad, dtype=jnp.int32) % n).at[dest].set(perm)
    inv_idx = jnp.zeros((n,), jnp.int32).at[perm].set(dest)
    return g_ids, idx_pad, inv_idx


def kernel(hidden_states, token_type_ids, W1, b1, W2, b2):
    b, s, d = hidden_states.shape
    e, _, ff = W1.shape
    n = b * s
    tm, tf = 512, 1024
    t_slots = n // tm + e

    flat = hidden_states.reshape(n, d)
    tt = token_type_ids.reshape(n).astype(jnp.int32)

    g_ids, idx_pad, inv_idx = _routing(tt, n, e, tm, t_slots)

    x_padded = _sc_row_gather(flat, idx_pad, chunk=32)
    y_padded = _grouped_ffn(x_padded, W1, b1, W2, b2, g_ids, tm, tf)
    out = _sc_row_gather(y_padded, inv_idx, chunk=32)
    return out.reshape(b, s, d)


# speedup vs baseline: 1.2080x; 1.0256x over previous
import functools

import jax
import jax.numpy as jnp
from jax import lax
from jax.experimental import pallas as pl
from jax.experimental.pallas import tpu as pltpu
from jax.experimental.pallas import tpu_sc as plsc


def _sc_row_gather(table, idx, chunk):
    _, d = table.shape
    m = idx.shape[0]
    info = plsc.get_sparse_core_info()
    nw = info.num_cores * info.num_subcores
    rows_per_w = m // nw
    assert rows_per_w * nw == m
    n_chunks = rows_per_w // chunk
    assert n_chunks * chunk == rows_per_w
    assert chunk % 8 == 0 and chunk <= 128

    mesh = plsc.VectorSubcoreMesh(core_axis_name="c", subcore_axis_name="s")

    @functools.partial(
        pl.kernel,
        out_type=jax.ShapeDtypeStruct((m, d), table.dtype),
        mesh=mesh,
        scratch_types=[
            pltpu.VMEM((chunk,), jnp.int32),
            pltpu.VMEM((chunk, d), table.dtype),
            pltpu.SemaphoreType.DMA,
        ],
    )
    def gather_kernel(table_hbm, idx_hbm, out_hbm, idx_v, rows_v, sem):
        wid = lax.axis_index("s") * info.num_cores + lax.axis_index("c")
        base = wid * rows_per_w

        def body(c, carry):
            row0 = base + c * chunk
            pltpu.sync_copy(idx_hbm.at[pl.ds(row0, chunk)], idx_v)
            pltpu.async_copy(table_hbm.at[idx_v], rows_v, sem).wait()
            pltpu.sync_copy(rows_v, out_hbm.at[pl.ds(row0, chunk), :])
            return carry

        lax.fori_loop(0, n_chunks, body, 0)

    return gather_kernel(table, idx)


def _ffn_body(g_ref, m_ref, v_ref, x_ref, w1_ref, b1_ref, w2_ref, b2_ref,
              o_ref, *, kf_total):
    t = pl.program_id(0)
    kf = pl.program_id(1)

    @pl.when(v_ref[t] == 1)
    def _compute():
        x = x_ref[...]
        h = jnp.dot(x, w1_ref[0], preferred_element_type=jnp.float32)
        h = jax.nn.gelu(h + b1_ref[0, 0][None, :])
        acc = jnp.dot(h, w2_ref[0], preferred_element_type=jnp.float32)
        last = (kf == kf_total - 1).astype(jnp.float32)
        acc = acc + last * b2_ref[0, 0][None, :]

        @pl.when(kf == 0)
        def _store():
            o_ref[...] = acc

        @pl.when(kf != 0)
        def _accum():
            o_ref[...] += acc


def _grouped_ffn(x_padded, w1, b1, w2, b2, g_ids, m_ids, valid, tm, tf,
                 interpret=False):
    n_pad, d = x_padded.shape
    e, _, ff = w1.shape
    t_slots = g_ids.shape[0]
    kf_total = ff // tf
    assert t_slots * tm == n_pad

    grid_spec = pltpu.PrefetchScalarGridSpec(
        num_scalar_prefetch=3,
        grid=(t_slots, kf_total),
        in_specs=[
            pl.BlockSpec((tm, d), lambda t, kf, g, m, v: (m[t], 0)),
            pl.BlockSpec((1, d, tf), lambda t, kf, g, m, v: (g[t], 0, kf)),
            pl.BlockSpec((1, 1, tf), lambda t, kf, g, m, v: (g[t], 0, kf)),
            pl.BlockSpec((1, tf, d), lambda t, kf, g, m, v: (g[t], kf, 0)),
            pl.BlockSpec((1, 1, d), lambda t, kf, g, m, v: (g[t], 0, 0)),
        ],
        out_specs=pl.BlockSpec((tm, d), lambda t, kf, g, m, v: (m[t], 0)),
    )
    return pl.pallas_call(
        functools.partial(_ffn_body, kf_total=kf_total),
        grid_spec=grid_spec,
        out_shape=jax.ShapeDtypeStruct((n_pad, d), x_padded.dtype),
        compiler_params=pltpu.CompilerParams(
            dimension_semantics=("arbitrary", "arbitrary"),
        ),
        interpret=interpret,
    )(g_ids, m_ids, valid, x_padded, w1,
      b1.reshape(e, 1, ff), w2, b2.reshape(e, 1, d))


def _routing(tt, n, e, tm, t_slots):
    perm = jnp.argsort(tt).astype(jnp.int32)
    counts = jnp.sum(tt[None, :] == jnp.arange(e, dtype=jnp.int32)[:, None],
                     axis=1, dtype=jnp.int32)
    offsets = jnp.concatenate([jnp.zeros((1,), jnp.int32),
                               jnp.cumsum(counts).astype(jnp.int32)])
    ntiles = (counts + tm - 1) // tm
    base = jnp.concatenate([jnp.zeros((1,), jnp.int32),
                            jnp.cumsum(ntiles).astype(jnp.int32)])
    pad_off = base * tm

    ts = jnp.arange(t_slots, dtype=jnp.int32)
    g = jnp.searchsorted(base, ts, side="right").astype(jnp.int32) - 1
    g_ids = jnp.clip(g, 0, e - 1)
    n_used = base[-1]
    valid = (ts < n_used).astype(jnp.int32)
    m_ids = jnp.minimum(ts, jnp.maximum(n_used - 1, 0))

    i = jnp.arange(n, dtype=jnp.int32)
    e_i = jnp.searchsorted(offsets, i, side="right").astype(jnp.int32) - 1
    dest = pad_off[e_i] + (i - offsets[e_i])

    n_pad = t_slots * tm
    idx_pad = (jnp.arange(n_pad, dtype=jnp.int32) % n).at[dest].set(perm)
    inv_idx = jnp.zeros((n,), jnp.int32).at[perm].set(dest)
    return g_ids, m_ids, valid, idx_pad, inv_idx


def kernel(hidden_states, token_type_ids, W1, b1, W2, b2):
    b, s, d = hidden_states.shape
    e, _, ff = W1.shape
    n = b * s
    tm, tf = 512, 1024
    t_slots = n // tm + e

    flat = hidden_states.reshape(n, d)
    tt = token_type_ids.reshape(n).astype(jnp.int32)

    g_ids, m_ids, valid, idx_pad, inv_idx = _routing(tt, n, e, tm, t_slots)

    x_padded = _sc_row_gather(flat, idx_pad, chunk=32)
    y_padded = _grouped_ffn(x_padded, W1, b1, W2, b2,
                            g_ids, m_ids, valid, tm, tf)
    out = _sc_row_gather(y_padded, inv_idx, chunk=32)
    return out.reshape(b, s, d)


# speedup vs baseline: 1.3574x; 1.1237x over previous
import functools

import jax
import jax.numpy as jnp
from jax import lax
from jax.experimental import pallas as pl
from jax.experimental.pallas import tpu as pltpu
from jax.experimental.pallas import tpu_sc as plsc


def _sc_row_gather(table, idx, chunk):
    _, d = table.shape
    m = idx.shape[0]
    info = plsc.get_sparse_core_info()
    nw = info.num_cores * info.num_subcores
    rows_per_w = m // nw
    assert rows_per_w * nw == m
    n_chunks = rows_per_w // chunk
    assert n_chunks * chunk == rows_per_w
    assert chunk % 8 == 0 and chunk <= 128

    mesh = plsc.VectorSubcoreMesh(core_axis_name="c", subcore_axis_name="s")

    @functools.partial(
        pl.kernel,
        out_type=jax.ShapeDtypeStruct((m, d), table.dtype),
        mesh=mesh,
        scratch_types=[
            pltpu.VMEM((chunk,), jnp.int32),
            pltpu.VMEM((chunk, d), table.dtype),
            pltpu.SemaphoreType.DMA,
        ],
    )
    def gather_kernel(table_hbm, idx_hbm, out_hbm, idx_v, rows_v, sem):
        wid = lax.axis_index("s") * info.num_cores + lax.axis_index("c")
        base = wid * rows_per_w

        def body(c, carry):
            row0 = base + c * chunk
            pltpu.sync_copy(idx_hbm.at[pl.ds(row0, chunk)], idx_v)
            pltpu.async_copy(table_hbm.at[idx_v], rows_v, sem).wait()
            pltpu.sync_copy(rows_v, out_hbm.at[pl.ds(row0, chunk), :])
            return carry

        lax.fori_loop(0, n_chunks, body, 0)

    return gather_kernel(table, idx)


def _ffn_body(g_ref, m_ref, v_ref, x_ref, w1_ref, b1_ref, w2_ref, b2_ref,
              o_ref, *, kf_total):
    t = pl.program_id(0)
    kf = pl.program_id(1)

    @pl.when(v_ref[t] == 1)
    def _compute():
        x = x_ref[...]
        h = jnp.dot(x, w1_ref[0], preferred_element_type=jnp.float32)
        h = jax.nn.gelu(h + b1_ref[0, 0][None, :])
        acc = jnp.dot(h, w2_ref[0], preferred_element_type=jnp.float32)
        last = (kf == kf_total - 1).astype(jnp.float32)
        acc = acc + last * b2_ref[0, 0][None, :]

        @pl.when(kf == 0)
        def _store():
            o_ref[...] = acc

        @pl.when(kf != 0)
        def _accum():
            o_ref[...] += acc


def _grouped_ffn(x_padded, w1, b1, w2, b2, g_ids, m_ids, valid, tm, tf,
                 interpret=False):
    n_pad, d = x_padded.shape
    e, _, ff = w1.shape
    t_slots = g_ids.shape[0]
    kf_total = ff // tf
    assert t_slots * tm == n_pad

    def kf_sel(kf, v, t):
        return jnp.where(v[t] == 1, kf, kf_total - 1)

    grid_spec = pltpu.PrefetchScalarGridSpec(
        num_scalar_prefetch=3,
        grid=(t_slots, kf_total),
        in_specs=[
            pl.BlockSpec((tm, d), lambda t, kf, g, m, v: (m[t], 0)),
            pl.BlockSpec((1, d, tf),
                         lambda t, kf, g, m, v: (g[t], 0, kf_sel(kf, v, t))),
            pl.BlockSpec((1, 1, tf),
                         lambda t, kf, g, m, v: (g[t], 0, kf_sel(kf, v, t))),
            pl.BlockSpec((1, tf, d),
                         lambda t, kf, g, m, v: (g[t], kf_sel(kf, v, t), 0)),
            pl.BlockSpec((1, 1, d), lambda t, kf, g, m, v: (g[t], 0, 0)),
        ],
        out_specs=pl.BlockSpec((tm, d), lambda t, kf, g, m, v: (m[t], 0)),
    )
    return pl.pallas_call(
        functools.partial(_ffn_body, kf_total=kf_total),
        grid_spec=grid_spec,
        out_shape=jax.ShapeDtypeStruct((n_pad, d), x_padded.dtype),
        compiler_params=pltpu.CompilerParams(
            dimension_semantics=("arbitrary", "arbitrary"),
        ),
        interpret=interpret,
    )(g_ids, m_ids, valid, x_padded, w1,
      b1.reshape(e, 1, ff), w2, b2.reshape(e, 1, d))


def _routing(tt, n, e, tm, t_slots):
    perm = jnp.argsort(tt).astype(jnp.int32)
    counts = jnp.sum(tt[None, :] == jnp.arange(e, dtype=jnp.int32)[:, None],
                     axis=1, dtype=jnp.int32)
    offsets = jnp.concatenate([jnp.zeros((1,), jnp.int32),
                               jnp.cumsum(counts).astype(jnp.int32)])
    ntiles = (counts + tm - 1) // tm
    base = jnp.concatenate([jnp.zeros((1,), jnp.int32),
                            jnp.cumsum(ntiles).astype(jnp.int32)])
    pad_off = base * tm

    ts = jnp.arange(t_slots, dtype=jnp.int32)
    g = jnp.searchsorted(base, ts, side="right").astype(jnp.int32) - 1
    g_ids = jnp.clip(g, 0, e - 1)
    n_used = base[-1]
    valid = (ts < n_used).astype(jnp.int32)
    m_ids = jnp.minimum(ts, jnp.maximum(n_used - 1, 0))

    i = jnp.arange(n, dtype=jnp.int32)
    e_i = jnp.searchsorted(offsets, i, side="right").astype(jnp.int32) - 1
    dest = pad_off[e_i] + (i - offsets[e_i])

    n_pad = t_slots * tm
    idx_pad = (jnp.arange(n_pad, dtype=jnp.int32) % n).at[dest].set(perm)
    inv_idx = jnp.zeros((n,), jnp.int32).at[perm].set(dest)
    return g_ids, m_ids, valid, idx_pad, inv_idx


def kernel(hidden_states, token_type_ids, W1, b1, W2, b2):
    b, s, d = hidden_states.shape
    e, _, ff = W1.shape
    n = b * s
    tm, tf = 512, 1024
    t_slots = n // tm + e

    flat = hidden_states.reshape(n, d)
    tt = token_type_ids.reshape(n).astype(jnp.int32)

    g_ids, m_ids, valid, idx_pad, inv_idx = _routing(tt, n, e, tm, t_slots)

    x_padded = _sc_row_gather(flat, idx_pad, chunk=32)
    y_padded = _grouped_ffn(x_padded, W1, b1, W2, b2,
                            g_ids, m_ids, valid, tm, tf)
    out = _sc_row_gather(y_padded, inv_idx, chunk=32)
    return out.reshape(b, s, d)


# speedup vs baseline: 1.4609x; 1.0762x over previous
import functools

import jax
import jax.numpy as jnp
from jax import lax
from jax.experimental import pallas as pl
from jax.experimental.pallas import tpu as pltpu
from jax.experimental.pallas import tpu_sc as plsc


def _sc_row_gather(table, idx, chunk):
    _, d = table.shape
    m = idx.shape[0]
    info = plsc.get_sparse_core_info()
    nw = info.num_cores * info.num_subcores
    rows_per_w = m // nw
    assert rows_per_w * nw == m
    n_chunks = rows_per_w // chunk
    assert n_chunks * chunk == rows_per_w
    assert chunk % 8 == 0 and chunk <= 128

    mesh = plsc.VectorSubcoreMesh(core_axis_name="c", subcore_axis_name="s")

    @functools.partial(
        pl.kernel,
        out_type=jax.ShapeDtypeStruct((m, d), table.dtype),
        mesh=mesh,
        scratch_types=[
            pltpu.VMEM((chunk,), jnp.int32),
            pltpu.VMEM((chunk, d), table.dtype),
            pltpu.SemaphoreType.DMA,
        ],
    )
    def gather_kernel(table_hbm, idx_hbm, out_hbm, idx_v, rows_v, sem):
        wid = lax.axis_index("s") * info.num_cores + lax.axis_index("c")
        base = wid * rows_per_w

        def body(c, carry):
            row0 = base + c * chunk
            pltpu.sync_copy(idx_hbm.at[pl.ds(row0, chunk)], idx_v)
            pltpu.async_copy(table_hbm.at[idx_v], rows_v, sem).wait()
            pltpu.sync_copy(rows_v, out_hbm.at[pl.ds(row0, chunk), :])
            return carry

        lax.fori_loop(0, n_chunks, body, 0)

    return gather_kernel(table, idx)


def _sc_row_scatter(rows, dest, n_out, chunk):
    n, d = rows.shape
    info = plsc.get_sparse_core_info()
    nw = info.num_cores * info.num_subcores
    rows_per_w = n // nw
    assert rows_per_w * nw == n
    n_chunks = rows_per_w // chunk
    assert n_chunks * chunk == rows_per_w
    assert chunk % 8 == 0 and chunk <= 128

    mesh = plsc.VectorSubcoreMesh(core_axis_name="c", subcore_axis_name="s")

    @functools.partial(
        pl.kernel,
        out_type=jax.ShapeDtypeStruct((n_out, d), rows.dtype),
        mesh=mesh,
        scratch_types=[
            pltpu.VMEM((chunk,), jnp.int32),
            pltpu.VMEM((chunk, d), rows.dtype),
            pltpu.SemaphoreType.DMA,
        ],
    )
    def scatter_kernel(rows_hbm, dest_hbm, out_hbm, idx_v, rows_v, sem):
        wid = lax.axis_index("s") * info.num_cores + lax.axis_index("c")
        base = wid * rows_per_w

        def body(c, carry):
            row0 = base + c * chunk
            pltpu.sync_copy(dest_hbm.at[pl.ds(row0, chunk)], idx_v)
            pltpu.sync_copy(rows_hbm.at[pl.ds(row0, chunk), :], rows_v)
            pltpu.async_copy(rows_v, out_hbm.at[idx_v], sem).wait()
            return carry

        lax.fori_loop(0, n_chunks, body, 0)

    return scatter_kernel(rows, dest)


def _ffn_body(g_ref, m_ref, v_ref, x_ref, w1_ref, b1_ref, w2_ref, b2_ref,
              o_ref, *, kf_total):
    t = pl.program_id(0)
    kf = pl.program_id(1)

    @pl.when(v_ref[t] == 1)
    def _compute():
        x = x_ref[...]
        h = jnp.dot(x, w1_ref[0], preferred_element_type=jnp.float32)
        h = jax.nn.gelu(h + b1_ref[0, 0][None, :])
        acc = jnp.dot(h, w2_ref[0], preferred_element_type=jnp.float32)
        last = (kf == kf_total - 1).astype(jnp.float32)
        acc = acc + last * b2_ref[0, 0][None, :]

        @pl.when(kf == 0)
        def _store():
            o_ref[...] = acc

        @pl.when(kf != 0)
        def _accum():
            o_ref[...] += acc


def _grouped_ffn(x_padded, w1, b1, w2, b2, g_ids, m_ids, valid, tm, tf,
                 interpret=False):
    n_pad, d = x_padded.shape
    e, _, ff = w1.shape
    t_slots = g_ids.shape[0]
    kf_total = ff // tf
    assert t_slots * tm == n_pad

    def kf_sel(kf, v, t):
        return jnp.where(v[t] == 1, kf, kf_total - 1)

    grid_spec = pltpu.PrefetchScalarGridSpec(
        num_scalar_prefetch=3,
        grid=(t_slots, kf_total),
        in_specs=[
            pl.BlockSpec((tm, d), lambda t, kf, g, m, v: (m[t], 0)),
            pl.BlockSpec((1, d, tf),
                         lambda t, kf, g, m, v: (g[t], 0, kf_sel(kf, v, t))),
            pl.BlockSpec((1, 1, tf),
                         lambda t, kf, g, m, v: (g[t], 0, kf_sel(kf, v, t))),
            pl.BlockSpec((1, tf, d),
                         lambda t, kf, g, m, v: (g[t], kf_sel(kf, v, t), 0)),
            pl.BlockSpec((1, 1, d), lambda t, kf, g, m, v: (g[t], 0, 0)),
        ],
        out_specs=pl.BlockSpec((tm, d), lambda t, kf, g, m, v: (m[t], 0)),
    )
    return pl.pallas_call(
        functools.partial(_ffn_body, kf_total=kf_total),
        grid_spec=grid_spec,
        out_shape=jax.ShapeDtypeStruct((n_pad, d), x_padded.dtype),
        compiler_params=pltpu.CompilerParams(
            dimension_semantics=("arbitrary", "arbitrary"),
        ),
        interpret=interpret,
    )(g_ids, m_ids, valid, x_padded, w1,
      b1.reshape(e, 1, ff), w2, b2.reshape(e, 1, d))


def _routing(tt, n, e, tm, t_slots):
    onehot = (tt[:, None] == jnp.arange(e, dtype=jnp.int32)[None, :])
    cum = jnp.cumsum(onehot.astype(jnp.int32), axis=0)
    rank = jnp.take_along_axis(cum, tt[:, None], axis=1)[:, 0] - 1
    counts = cum[-1]

    ntiles = (counts + tm - 1) // tm
    base = jnp.concatenate([jnp.zeros((1,), jnp.int32),
                            jnp.cumsum(ntiles).astype(jnp.int32)])
    pad_off = base * tm

    ts = jnp.arange(t_slots, dtype=jnp.int32)
    g = jnp.searchsorted(base, ts, side="right").astype(jnp.int32) - 1
    g_ids = jnp.clip(g, 0, e - 1)
    n_used = base[-1]
    valid = (ts < n_used).astype(jnp.int32)
    m_ids = jnp.minimum(ts, jnp.maximum(n_used - 1, 0))

    dest = pad_off[tt] + rank
    return g_ids, m_ids, valid, dest


def kernel(hidden_states, token_type_ids, W1, b1, W2, b2):
    b, s, d = hidden_states.shape
    e, _, ff = W1.shape
    n = b * s
    tm, tf = 512, 1024
    t_slots = n // tm + e

    flat = hidden_states.reshape(n, d)
    tt = token_type_ids.reshape(n).astype(jnp.int32)

    g_ids, m_ids, valid, dest = _routing(tt, n, e, tm, t_slots)

    x_padded = _sc_row_scatter(flat, dest, t_slots * tm, chunk=32)
    y_padded = _grouped_ffn(x_padded, W1, b1, W2, b2,
                            g_ids, m_ids, valid, tm, tf)
    out = _sc_row_gather(y_padded, dest, chunk=32)
    return out.reshape(b, s, d)
